# Initial kernel scaffold; baseline (speedup 1.0000x reference)
import functools

import jax
import jax.numpy as jnp
from jax import lax
from jax.experimental import pallas as pl
from jax.experimental.pallas import tpu as pltpu
from jax.experimental.pallas import tpu_sc as plsc

_B = 16384
_SPARSE_START = 13
_FIELD_NUM = 26
_VOCAB = 100000
_DIM = 16

_NC = 2
_NS = 16
_NW = _NC * _NS
_LANES = 16

_ROWS_PER_W = _B // _NW
_CH = 128
_NITER = _ROWS_PER_W // _CH


def _sc_embed(xs_t, x_dense_f, table_flat):
    out_cols = _SPARSE_START + _FIELD_NUM * _DIM

    mesh = plsc.VectorSubcoreMesh(core_axis_name="c", subcore_axis_name="s")

    @functools.partial(
        pl.kernel,
        mesh=mesh,
        out_type=jax.ShapeDtypeStruct((_B, out_cols), jnp.float32),
        scratch_types=[
            pltpu.VMEM((_FIELD_NUM, _CH), jnp.int32),
            pltpu.VMEM((_FIELD_NUM, _CH, _DIM), jnp.float32),
            pltpu.VMEM((_CH, _SPARSE_START), jnp.float32),
            pltpu.SemaphoreType.DMA,
            pltpu.SemaphoreType.DMA,
        ],
    )
    def k(xs_hbm, xd_hbm, tab_hbm, out_hbm, idx_v, emb_v, dsf_v, gsem, osem):
        wid = lax.axis_index("s") * _NC + lax.axis_index("c")
        w_base = wid * _ROWS_PER_W

        def chunk(it, carry):
            base = w_base + it * _CH

            pltpu.sync_copy(xs_hbm.at[:, pl.ds(base, _CH)], idx_v)
            pltpu.sync_copy(xd_hbm.at[pl.ds(base, _CH)], dsf_v)

            for f in range(_FIELD_NUM):
                bias = f * _VOCAB
                if bias:
                    for j in range(_CH // _LANES):
                        sl = pl.ds(j * _LANES, _LANES)
                        idx_v[f, sl] = idx_v[f, sl] + bias

            handles = []
            for f in range(_FIELD_NUM):
                handles.append(
                    pltpu.async_copy(tab_hbm.at[idx_v.at[f]], emb_v.at[f], gsem)
                )

            od = pltpu.async_copy(
                dsf_v, out_hbm.at[pl.ds(base, _CH), pl.ds(0, _SPARSE_START)], osem
            )

            outs = []
            for f in range(_FIELD_NUM):
                handles[f].wait()
                col = _SPARSE_START + f * _DIM
                outs.append(
                    pltpu.async_copy(
                        emb_v.at[f],
                        out_hbm.at[pl.ds(base, _CH), pl.ds(col, _DIM)],
                        osem,
                    )
                )

            od.wait()
            for o in outs:
                o.wait()
            return carry

        lax.fori_loop(0, _NITER, chunk, 0)

    return k(xs_t, x_dense_f, table_flat)


def kernel(X, tables):
    x_dense_f = X[:, :_SPARSE_START].astype(jnp.float32)
    xs_t = X[:, _SPARSE_START:].T
    table_flat = tables.reshape(_FIELD_NUM * _VOCAB, _DIM)
    return _sc_embed(xs_t, x_dense_f, table_flat)


# speedup vs baseline: 1.1075x; 1.1075x over previous
import functools

import jax
import jax.numpy as jnp
from jax import lax
from jax.experimental import pallas as pl
from jax.experimental.pallas import tpu as pltpu
from jax.experimental.pallas import tpu_sc as plsc

_B = 16384
_SPARSE_START = 13
_FIELD_NUM = 26
_VOCAB = 100000
_DIM = 16

_NC = 2
_NS = 16
_NW = _NC * _NS
_LANES = 16

_ROWS_PER_W = _B // _NW
_CH = 128
_NITER = _ROWS_PER_W // _CH

_OUT_COLS = _SPARSE_START + _FIELD_NUM * _DIM


def _sc_embed(xs_t, xd_pad, table_flat):
    mesh = plsc.VectorSubcoreMesh(core_axis_name="c", subcore_axis_name="s")

    @functools.partial(
        pl.kernel,
        mesh=mesh,
        compiler_params=pltpu.CompilerParams(use_tc_tiling_on_sc=False),
        out_type=jax.ShapeDtypeStruct((_B, _OUT_COLS), jnp.float32),
        scratch_types=[
            pltpu.VMEM((_FIELD_NUM, _CH), jnp.int32),
            pltpu.VMEM((_FIELD_NUM, _CH, _DIM), jnp.float32),
            pltpu.VMEM((_CH, _OUT_COLS), jnp.float32),
            pltpu.SemaphoreType.DMA,
            pltpu.SemaphoreType.DMA,
        ],
    )
    def k(xs_hbm, xd_hbm, tab_hbm, out_hbm, idx_v, emb_v, row_v, gsem, dsem):
        wid = lax.axis_index("s") * _NC + lax.axis_index("c")
        w_base = wid * _ROWS_PER_W

        def chunk(it, carry):
            base = pl.multiple_of(w_base + it * _CH, _CH)

            pltpu.sync_copy(xs_hbm.at[:, pl.ds(base, _CH)], idx_v)

            dh = pltpu.async_copy(
                xd_hbm.at[pl.ds(base, _CH)],
                row_v.at[:, pl.ds(0, 16)],
                dsem,
            )

            for f in range(_FIELD_NUM):
                bias = f * _VOCAB
                if bias:
                    for j in range(_CH // _LANES):
                        sl = pl.ds(j * _LANES, _LANES)
                        idx_v[f, sl] = idx_v[f, sl] + bias

            handles = []
            for f in range(_FIELD_NUM):
                handles.append(
                    pltpu.async_copy(tab_hbm.at[idx_v.at[f]], emb_v.at[f], gsem)
                )
            dh.wait()
            for h in handles:
                h.wait()

            def put_row(r, c):
                for f in range(_FIELD_NUM):
                    row_v[r, pl.ds(_SPARSE_START + f * _DIM, _DIM)] = (
                        emb_v[f, r, pl.ds(0, _DIM)]
                    )
                return c

            lax.fori_loop(0, _CH, put_row, 0)

            pltpu.sync_copy(row_v, out_hbm.at[pl.ds(base, _CH)])
            return carry

        lax.fori_loop(0, _NITER, chunk, 0)

    return k(xs_t, xd_pad, table_flat)


def kernel(X, tables):
    xd_pad = jnp.pad(
        X[:, :_SPARSE_START].astype(jnp.float32), ((0, 0), (0, 16 - _SPARSE_START))
    )
    xs_t = X[:, _SPARSE_START:].T
    table_flat = tables.reshape(_FIELD_NUM * _VOCAB, _DIM)
    return _sc_embed(xs_t, xd_pad, table_flat)


# speedup vs baseline: 1.1158x; 1.0076x over previous
import functools

import jax
import jax.numpy as jnp
from jax import lax
from jax.experimental import pallas as pl
from jax.experimental.pallas import tpu as pltpu
from jax.experimental.pallas import tpu_sc as plsc

_B = 16384
_SPARSE_START = 13
_FIELD_NUM = 26
_VOCAB = 100000
_DIM = 16
_XCOLS = _SPARSE_START + _FIELD_NUM

_NC = 2
_NS = 16
_NW = _NC * _NS
_LANES = 16

_ROWS_PER_W = _B // _NW
_CH = 128
_NITER = _ROWS_PER_W // _CH

_OUT_COLS = _SPARSE_START + _FIELD_NUM * _DIM


def _sc_embed(x_flat, table_flat):
    mesh = plsc.VectorSubcoreMesh(core_axis_name="c", subcore_axis_name="s")

    @functools.partial(
        pl.kernel,
        mesh=mesh,
        compiler_params=pltpu.CompilerParams(
            use_tc_tiling_on_sc=False, needs_layout_passes=False
        ),
        out_type=jax.ShapeDtypeStruct((_B * _OUT_COLS,), jnp.float32),
        scratch_types=[
            pltpu.VMEM((_CH * _XCOLS,), jnp.int32),
            pltpu.VMEM((_FIELD_NUM, _CH), jnp.int32),
            pltpu.VMEM((_FIELD_NUM, _CH, _DIM), jnp.float32),
            pltpu.VMEM((_CH * _OUT_COLS,), jnp.float32),
            pltpu.SemaphoreType.DMA,
        ],
    )
    def k(x_hbm, tab_hbm, out_hbm, xs_v, idx_v, emb_v, row_v, gsem):
        wid = lax.axis_index("s") * _NC + lax.axis_index("c")
        w_base = wid * _ROWS_PER_W

        def chunk(it, carry):
            base = pl.multiple_of(w_base + it * _CH, _CH)

            pltpu.sync_copy(x_hbm.at[pl.ds(base * _XCOLS, _CH * _XCOLS)], xs_v)

            row_addr = lax.iota(jnp.int32, _LANES) * _XCOLS
            for f in range(_FIELD_NUM):
                bias = f * _VOCAB
                for j in range(_CH // _LANES):
                    addr = row_addr + (j * _LANES * _XCOLS + _SPARSE_START + f)
                    vals = plsc.load_gather(xs_v, [addr])
                    idx_v[f, pl.ds(j * _LANES, _LANES)] = vals + bias

            handles = []
            for f in range(_FIELD_NUM):
                handles.append(
                    pltpu.async_copy(tab_hbm.at[idx_v.at[f]], emb_v.at[f], gsem)
                )
            for h in handles:
                h.wait()

            def put_row(r, c):
                d = xs_v[pl.ds(r * _XCOLS, _LANES)].astype(jnp.float32)
                row_v[pl.ds(r * _OUT_COLS, _LANES)] = d
                for f in range(_FIELD_NUM):
                    row_v[pl.ds(r * _OUT_COLS + _SPARSE_START + f * _DIM, _DIM)] = (
                        emb_v[f, r, pl.ds(0, _DIM)]
                    )
                return c

            lax.fori_loop(0, _CH, put_row, 0)

            pltpu.sync_copy(
                row_v,
                out_hbm.at[pl.ds(pl.multiple_of(base * _OUT_COLS, 8), _CH * _OUT_COLS)],
            )
            return carry

        lax.fori_loop(0, _NITER, chunk, 0)

    return k(x_flat, table_flat)


def kernel(X, tables):
    x_flat = X.reshape(_B * _XCOLS)
    table_flat = tables.reshape(_FIELD_NUM * _VOCAB, _DIM)
    out_flat = _sc_embed(x_flat, table_flat)
    return out_flat.reshape(_B, _OUT_COLS)
